# baseline (speedup 1.0000x reference)
import jax
import jax.numpy as jnp
from jax import lax
from jax.experimental import pallas as pl
from jax.experimental.pallas import tpu as pltpu
from jax.experimental.pallas import tpu_sc as plsc

VOCAB = 1000000
EMBED_DIM = 32
B = 16384
L = 50

NUM_CORES = 2
NUM_SUBCORES = 16
NW = NUM_CORES * NUM_SUBCORES

NBT = B // 128
BT_PER_W = NBT // NW
ROWS_PER_W = 128 * BT_PER_W
N_ITEMS = L * BT_PER_W


def _gather_body(table_hbm, idxt_hbm, out_hbm, idx_v, g0, g1, t0, t1,
                 sem_g0, sem_g1, sem_s0, sem_s1):
    wid = lax.axis_index("s") * NUM_CORES + lax.axis_index("c")
    G = (g0, g1)
    T = (t0, t1)
    sg = (sem_g0, sem_g1)
    ss = (sem_s0, sem_s1)

    pltpu.sync_copy(idxt_hbm.at[:, pl.ds(wid * ROWS_PER_W, ROWS_PER_W)], idx_v)

    rows_k = [lax.iota(jnp.int32, 16) + 16 * k for k in range(8)]

    pltpu.async_copy(table_hbm.at[idx_v.at[0, pl.ds(0, 128)]], G[0], sg[0])

    @pl.loop(0, N_ITEMS, step=2)
    def _items(ibase):
        for b in range(2):
            i = ibase + b
            l = i // BT_PER_W
            t = i % BT_PER_W

            pltpu.make_async_copy(
                table_hbm.at[pl.ds(0, 128)], G[b], sg[b]).wait()

            @pl.when(i + 1 < N_ITEMS)
            def _fire_next():
                i1 = i + 1
                pltpu.async_copy(
                    table_hbm.at[
                        idx_v.at[i1 // BT_PER_W,
                                 pl.ds(128 * (i1 % BT_PER_W), 128)]],
                    G[1 - b], sg[1 - b])

            @pl.when(i >= 2)
            def _drain_stores():
                for et in range(4):
                    pltpu.make_async_copy(
                        out_hbm.at[0, 0, 0], T[b].at[et], ss[b]).wait()

            for k in range(8):
                for e in range(EMBED_DIM):
                    col = jnp.full((16,), e, jnp.int32)
                    vec = plsc.load_gather(G[b], [rows_k[k], col])
                    T[b][e // 8, e % 8, pl.ds(16 * k, 16)] = vec

            for et in range(4):
                pltpu.async_copy(
                    T[b].at[et], out_hbm.at[l, et, BT_PER_W * wid + t], ss[b])

    for b in range(2):
        for et in range(4):
            pltpu.make_async_copy(
                out_hbm.at[0, 0, 0], T[b].at[et], ss[b]).wait()


_mesh = plsc.VectorSubcoreMesh(
    core_axis_name="c", subcore_axis_name="s",
    num_cores=NUM_CORES, num_subcores=NUM_SUBCORES,
)

_sc_gather = pl.kernel(
    _gather_body,
    out_type=jax.ShapeDtypeStruct((L, 4, NBT, 8, 128), jnp.float32),
    mesh=_mesh,
    scratch_types=[
        pltpu.VMEM((L, ROWS_PER_W), jnp.int32),
        pltpu.VMEM((128, EMBED_DIM), jnp.float32),
        pltpu.VMEM((128, EMBED_DIM), jnp.float32),
        pltpu.VMEM((4, 8, 128), jnp.float32),
        pltpu.VMEM((4, 8, 128), jnp.float32),
        pltpu.SemaphoreType.DMA,
        pltpu.SemaphoreType.DMA,
        pltpu.SemaphoreType.DMA,
        pltpu.SemaphoreType.DMA,
    ],
    compiler_params=pltpu.CompilerParams(
        use_tc_tiling_on_sc=False, needs_layout_passes=False),
)


@jax.jit
def kernel(inputs, table):
    idx_t = inputs.T
    out_p = _sc_gather(table, idx_t)
    return out_p.transpose(2, 4, 0, 1, 3).reshape(B, L, EMBED_DIM)


# speedup vs baseline: 1.7257x; 1.7257x over previous
import jax
import jax.numpy as jnp
from jax import lax
from jax.experimental import pallas as pl
from jax.experimental.pallas import tpu as pltpu
from jax.experimental.pallas import tpu_sc as plsc

VOCAB = 1000000
EMBED_DIM = 32
B = 16384
L = 50

NUM_CORES = 2
NUM_SUBCORES = 16
NW = NUM_CORES * NUM_SUBCORES

NBT = B // 128
BT_PER_W = NBT // NW
ROWS_PER_W = 128 * BT_PER_W
N_ITEMS = L * BT_PER_W


def _gather_body(table_hbm, idxt_hbm, out_hbm, idx_v, g0, g1, t0, t1,
                 sem_g0, sem_g1, sem_s0, sem_s1):
    wid = lax.axis_index("s") * NUM_CORES + lax.axis_index("c")
    G = (g0, g1)
    T = (t0, t1)
    sg = (sem_g0, sem_g1)
    ss = (sem_s0, sem_s1)

    pltpu.sync_copy(idxt_hbm.at[:, pl.ds(wid * ROWS_PER_W, ROWS_PER_W)], idx_v)

    iota = lax.iota(jnp.int32, 16)
    et_lo, ei_lo = iota >> 3, iota & 7
    et_hi, ei_hi = (iota + 16) >> 3, (iota + 16) & 7

    pltpu.async_copy(table_hbm.at[idx_v.at[0, pl.ds(0, 128)]], G[0], sg[0])

    @pl.loop(0, N_ITEMS, step=2)
    def _items(ibase):
        for b in range(2):
            i = ibase + b
            l = i // BT_PER_W
            t = i % BT_PER_W

            pltpu.make_async_copy(
                table_hbm.at[pl.ds(0, 128)], G[b], sg[b]).wait()

            @pl.when(i + 1 < N_ITEMS)
            def _fire_next():
                i1 = i + 1
                pltpu.async_copy(
                    table_hbm.at[
                        idx_v.at[i1 // BT_PER_W,
                                 pl.ds(128 * (i1 % BT_PER_W), 128)]],
                    G[1 - b], sg[1 - b])

            @pl.when(i >= 2)
            def _drain_stores():
                for et in range(4):
                    pltpu.make_async_copy(
                        out_hbm.at[0, 0, 0],
                        T[b].at[et, :, pl.ds(0, 128)], ss[b]).wait()

            for j in range(128):
                bi = jnp.full((16,), j, jnp.int32)
                va = G[b][j, pl.ds(0, 16)]
                vb = G[b][j, pl.ds(16, 16)]
                plsc.store_scatter(T[b], [et_lo, ei_lo, bi], va)
                plsc.store_scatter(T[b], [et_hi, ei_hi, bi], vb)

            for et in range(4):
                pltpu.async_copy(
                    T[b].at[et, :, pl.ds(0, 128)],
                    out_hbm.at[l, et, BT_PER_W * wid + t], ss[b])

    for b in range(2):
        for et in range(4):
            pltpu.make_async_copy(
                out_hbm.at[0, 0, 0],
                T[b].at[et, :, pl.ds(0, 128)], ss[b]).wait()


_mesh = plsc.VectorSubcoreMesh(
    core_axis_name="c", subcore_axis_name="s",
    num_cores=NUM_CORES, num_subcores=NUM_SUBCORES,
)

_sc_gather = pl.kernel(
    _gather_body,
    out_type=jax.ShapeDtypeStruct((L, 4, NBT, 8, 128), jnp.float32),
    mesh=_mesh,
    scratch_types=[
        pltpu.VMEM((L, ROWS_PER_W), jnp.int32),
        pltpu.VMEM((128, EMBED_DIM), jnp.float32),
        pltpu.VMEM((128, EMBED_DIM), jnp.float32),
        pltpu.VMEM((4, 8, 129), jnp.float32),
        pltpu.VMEM((4, 8, 129), jnp.float32),
        pltpu.SemaphoreType.DMA,
        pltpu.SemaphoreType.DMA,
        pltpu.SemaphoreType.DMA,
        pltpu.SemaphoreType.DMA,
    ],
    compiler_params=pltpu.CompilerParams(
        use_tc_tiling_on_sc=False, needs_layout_passes=False),
)


@jax.jit
def kernel(inputs, table):
    idx_t = inputs.T
    out_p = _sc_gather(table, idx_t)
    return out_p.transpose(2, 4, 0, 1, 3).reshape(B, L, EMBED_DIM)
